# baseline (speedup 1.0000x reference)
import jax
import jax.numpy as jnp
import numpy as np
from jax.experimental import pallas as pl

_ROWS = 32
_V = 1000000
_SUB = 8
_LANES = _V // _SUB

_KS0 = np.uint32(0)
_KS1 = np.uint32(1)
_KS2 = np.uint32(0x1BD11BDB)
_ROT_A = (13, 15, 26, 6)
_ROT_B = (17, 29, 16, 24)


def _threefry_bits(j):
    x0 = jnp.zeros_like(j) + _KS0
    x1 = j + _KS1
    sched = ((_ROT_A, _KS1, _KS2, 1), (_ROT_B, _KS2, _KS0, 2),
             (_ROT_A, _KS0, _KS1, 3), (_ROT_B, _KS1, _KS2, 4),
             (_ROT_A, _KS2, _KS0, 5))
    for rots, ka, kb, c in sched:
        for r in rots:
            x0 = x0 + x1
            x1 = (x1 << np.uint32(r)) | (x1 >> np.uint32(32 - r))
            x1 = x0 ^ x1
        x0 = x0 + ka
        x1 = x1 + kb + np.uint32(c)
    return x0 ^ x1


def _body(x_ref, p_ref, s_ref):
    r = pl.program_id(0)
    x = x_ref[0]
    j = (jax.lax.broadcasted_iota(jnp.int32, x.shape, 0) * _LANES
         + jax.lax.broadcasted_iota(jnp.int32, x.shape, 1)
         + r * _V)
    bits = _threefry_bits(j.astype(jnp.uint32))
    fb = (bits >> np.uint32(9)) | np.uint32(0x3F800000)
    u = jnp.maximum(np.float32(0),
                    jax.lax.bitcast_convert_type(fb, jnp.float32)
                    - np.float32(1))
    q = -jnp.log1p(-u)
    m = jnp.max(x)
    e = jnp.exp(x - m)
    s = jnp.sum(e)
    p = e / s
    p_ref[0] = p
    t = p / q
    mt = jnp.max(t)
    flat = (jax.lax.broadcasted_iota(jnp.int32, t.shape, 0) * _LANES
            + jax.lax.broadcasted_iota(jnp.int32, t.shape, 1))
    idx = jnp.min(jnp.where(t == mt, flat, _V))
    s_ref[0] = jnp.full((1, 128), idx, jnp.int32)


def kernel(logits):
    x3 = logits.reshape(_ROWS, _SUB, _LANES)
    row_spec = pl.BlockSpec((1, _SUB, _LANES), lambda i: (i, 0, 0))
    probs, samp = pl.pallas_call(
        _body,
        grid=(_ROWS,),
        in_specs=[row_spec],
        out_specs=[row_spec,
                   pl.BlockSpec((1, 1, 128), lambda i: (i, 0, 0))],
        out_shape=[jax.ShapeDtypeStruct((_ROWS, _SUB, _LANES), jnp.float32),
                   jax.ShapeDtypeStruct((_ROWS, 1, 128), jnp.int32)],
    )(x3)
    return probs.reshape(_ROWS, _V), samp[:, 0, 0]
